# Initial kernel scaffold; baseline (speedup 1.0000x reference)
import jax
import jax.numpy as jnp
from jax.experimental import pallas as pl


def kernel(features, edge_index, W1, b1, W2, b2, W3, b3):
    raise NotImplementedError("write your pallas kernel here")


# speedup vs baseline: 4.8256x; 4.8256x over previous
import functools

import jax
import jax.numpy as jnp
from jax import lax
from jax.experimental import pallas as pl
from jax.experimental.pallas import tpu as pltpu
from jax.experimental.pallas import tpu_sc as plsc

N_NODES = 10000
N_EDGES = 160000
NC, NS = 2, 16
NW = NC * NS
K = 40
CH = N_EDGES // (NW * K)
NA = 10112
RPT = NA // NS

BM = 1000


def _worker(c, s):
    return s * NC + c


def _deg_body(dst_hbm, ones_hbm, zeros_hbm, out_hbm, idx_v, ones_v, accum, sem):
    del sem
    c = lax.axis_index("c")
    s = lax.axis_index("s")
    wid = _worker(c, s)
    pltpu.sync_copy(dst_hbm.at[wid], idx_v)
    pltpu.sync_copy(ones_hbm, ones_v)
    pltpu.sync_copy(zeros_hbm.at[pl.ds(s * RPT, RPT)],
                    accum.at[pl.ds(s * RPT, RPT)])
    plsc.subcore_barrier()

    @pl.loop(0, CH)
    def _chunk(ch):
        pltpu.sync_copy(ones_v, accum.at[idx_v.at[ch]], add=True)

    plsc.subcore_barrier()
    pltpu.sync_copy(accum.at[pl.ds(s * RPT, RPT)],
                    out_hbm.at[c, pl.ds(s * RPT, RPT)])


_deg_kernel = pl.kernel(
    _deg_body,
    out_type=jax.ShapeDtypeStruct((NC, NA, 16), jnp.float32),
    mesh=plsc.VectorSubcoreMesh(core_axis_name="c", subcore_axis_name="s",
                                num_cores=NC, num_subcores=NS),
    compiler_params=pltpu.CompilerParams(use_tc_tiling_on_sc=False),
    scratch_types=[
        pltpu.VMEM((CH, K), jnp.int32),
        pltpu.VMEM((K, 16), jnp.float32),
        pltpu.VMEM_SHARED((NA, 16), jnp.float32),
        pltpu.SemaphoreType.DMA,
    ],
)


def _prop_body(fw, y_hbm, src_hbm, dst_hbm, zeros_hbm, out_hbm,
               src_v, dst_v, buf, accum, sem):
    c = lax.axis_index("c")
    s = lax.axis_index("s")
    wid = _worker(c, s)
    pltpu.sync_copy(src_hbm.at[wid], src_v)
    pltpu.sync_copy(dst_hbm.at[wid], dst_v)
    pltpu.sync_copy(zeros_hbm.at[pl.ds(s * RPT, RPT)],
                    accum.at[pl.ds(s * RPT, RPT)])
    plsc.subcore_barrier()

    @pl.loop(0, CH)
    def _chunk(ch):
        pltpu.async_copy(y_hbm.at[src_v.at[ch]], buf, sem).wait()
        pltpu.sync_copy(buf, accum.at[dst_v.at[ch]], add=True)

    plsc.subcore_barrier()
    pltpu.sync_copy(accum.at[pl.ds(s * RPT, RPT)],
                    out_hbm.at[c, pl.ds(s * RPT, RPT)])


def _make_prop(fw):
    return pl.kernel(
        functools.partial(_prop_body, fw),
        out_type=jax.ShapeDtypeStruct((NC, NA, fw), jnp.float32),
        mesh=plsc.VectorSubcoreMesh(core_axis_name="c", subcore_axis_name="s",
                                    num_cores=NC, num_subcores=NS),
        compiler_params=pltpu.CompilerParams(use_tc_tiling_on_sc=False),
        scratch_types=[
            pltpu.VMEM((CH, K), jnp.int32),
            pltpu.VMEM((CH, K), jnp.int32),
            pltpu.VMEM((K, fw), jnp.float32),
            pltpu.VMEM_SHARED((NA, fw), jnp.float32),
            pltpu.SemaphoreType.DMA,
        ],
    )


_prop128 = _make_prop(128)
_prop48 = _make_prop(48)


def _norm_from(degp_ref):
    deg = degp_ref[0, :, 0] + degp_ref[1, :, 0]
    return lax.rsqrt(jnp.maximum(deg, 1.0))


def _mm1_body(x_ref, w_ref, degp_ref, ya_ref, yb_ref):
    norm = _norm_from(degp_ref)
    y = jnp.dot(x_ref[...], w_ref[...],
                preferred_element_type=jnp.float32) * norm[:, None]
    ya_ref[...] = y[:, :128]
    yb_ref[...] = y[:, 128:]


_mm1 = pl.pallas_call(
    _mm1_body,
    grid=(N_NODES // BM,),
    in_specs=[
        pl.BlockSpec((BM, 256), lambda i: (i, 0)),
        pl.BlockSpec((256, 256), lambda i: (0, 0)),
        pl.BlockSpec((2, BM, 16), lambda i: (0, i, 0)),
    ],
    out_specs=[
        pl.BlockSpec((BM, 128), lambda i: (i, 0)),
        pl.BlockSpec((BM, 128), lambda i: (i, 0)),
    ],
    out_shape=[
        jax.ShapeDtypeStruct((N_NODES, 128), jnp.float32),
        jax.ShapeDtypeStruct((N_NODES, 128), jnp.float32),
    ],
)


def _mm2_body(pa_ref, pb_ref, degp_ref, b_ref, w_ref, ya_ref, yb_ref):
    norm = _norm_from(degp_ref)
    nc = norm[:, None]
    ha = jnp.maximum((pa_ref[0] + pa_ref[1]) * nc + b_ref[0, :128], 0.0)
    hb = jnp.maximum((pb_ref[0] + pb_ref[1]) * nc + b_ref[0, 128:], 0.0)
    y = (jnp.dot(ha, w_ref[:128, :], preferred_element_type=jnp.float32)
         + jnp.dot(hb, w_ref[128:, :], preferred_element_type=jnp.float32)) * nc
    ya_ref[...] = y[:, :128]
    yb_ref[...] = y[:, 128:]


_mm2 = pl.pallas_call(
    _mm2_body,
    grid=(N_NODES // BM,),
    in_specs=[
        pl.BlockSpec((2, BM, 128), lambda i: (0, i, 0)),
        pl.BlockSpec((2, BM, 128), lambda i: (0, i, 0)),
        pl.BlockSpec((2, BM, 16), lambda i: (0, i, 0)),
        pl.BlockSpec((1, 256), lambda i: (0, 0)),
        pl.BlockSpec((256, 256), lambda i: (0, 0)),
    ],
    out_specs=[
        pl.BlockSpec((BM, 128), lambda i: (i, 0)),
        pl.BlockSpec((BM, 128), lambda i: (i, 0)),
    ],
    out_shape=[
        jax.ShapeDtypeStruct((N_NODES, 128), jnp.float32),
        jax.ShapeDtypeStruct((N_NODES, 128), jnp.float32),
    ],
)


def _mm3_body(pa_ref, pb_ref, degp_ref, b_ref, w_ref, y_ref):
    norm = _norm_from(degp_ref)
    nc = norm[:, None]
    ha = jnp.maximum((pa_ref[0] + pa_ref[1]) * nc + b_ref[0, :128], 0.0)
    hb = jnp.maximum((pb_ref[0] + pb_ref[1]) * nc + b_ref[0, 128:], 0.0)
    y_ref[...] = (jnp.dot(ha, w_ref[:128, :], preferred_element_type=jnp.float32)
                  + jnp.dot(hb, w_ref[128:, :],
                            preferred_element_type=jnp.float32)) * nc


_mm3 = pl.pallas_call(
    _mm3_body,
    grid=(N_NODES // BM,),
    in_specs=[
        pl.BlockSpec((2, BM, 128), lambda i: (0, i, 0)),
        pl.BlockSpec((2, BM, 128), lambda i: (0, i, 0)),
        pl.BlockSpec((2, BM, 16), lambda i: (0, i, 0)),
        pl.BlockSpec((1, 256), lambda i: (0, 0)),
        pl.BlockSpec((256, 48), lambda i: (0, 0)),
    ],
    out_specs=pl.BlockSpec((BM, 48), lambda i: (i, 0)),
    out_shape=jax.ShapeDtypeStruct((N_NODES, 48), jnp.float32),
)


def _final_body(p_ref, degp_ref, b_ref, o_ref):
    norm = _norm_from(degp_ref)
    o_ref[...] = ((p_ref[0, :, :40] + p_ref[1, :, :40]) * norm[:, None]
                  + b_ref[0])


_final = pl.pallas_call(
    _final_body,
    grid=(N_NODES // BM,),
    in_specs=[
        pl.BlockSpec((2, BM, 48), lambda i: (0, i, 0)),
        pl.BlockSpec((2, BM, 16), lambda i: (0, i, 0)),
        pl.BlockSpec((1, 40), lambda i: (0, 0)),
    ],
    out_specs=pl.BlockSpec((BM, 40), lambda i: (i, 0)),
    out_shape=jax.ShapeDtypeStruct((N_NODES, 40), jnp.float32),
)


def kernel(features, edge_index, W1, b1, W2, b2, W3, b3):
    src = edge_index[0].reshape(NW, CH, K)
    dst = edge_index[1].reshape(NW, CH, K)
    ones8 = jnp.ones((K, 16), jnp.float32)
    zeros8 = jnp.zeros((NA, 16), jnp.float32)
    zeros128 = jnp.zeros((NA, 128), jnp.float32)
    zeros48 = jnp.zeros((NA, 48), jnp.float32)
    W3p = jnp.pad(W3, ((0, 0), (0, 48 - W3.shape[1])))
    b1r = b1.reshape(1, 256)
    b2r = b2.reshape(1, 256)
    b3r = b3.reshape(1, 40)

    degp = _deg_kernel(dst, ones8, zeros8)
    y1a, y1b = _mm1(features, W1, degp)
    p1a = _prop128(y1a, src, dst, zeros128)
    p1b = _prop128(y1b, src, dst, zeros128)
    y2a, y2b = _mm2(p1a, p1b, degp, b1r, W2)
    p2a = _prop128(y2a, src, dst, zeros128)
    p2b = _prop128(y2b, src, dst, zeros128)
    y3 = _mm3(p2a, p2b, degp, b2r, W3p)
    p3 = _prop48(y3, src, dst, zeros48)
    return _final(p3, degp, b3r)
